# baseline (speedup 1.0000x reference)
import functools

import jax
import jax.numpy as jnp
from jax import lax
from jax.experimental import pallas as pl
from jax.experimental.pallas import tpu as pltpu
from jax.experimental.pallas import tpu_sc as plsc

N_FACTORS = 32

_info = plsc.get_sparse_core_info()
_NC, _NS, _L = _info.num_cores, _info.num_subcores, _info.num_lanes
_NW = _NC * _NS


@functools.partial(jax.jit, static_argnames=("batch",))
def _run(x, user_emb, movie_emb, batch):
    b_per_w = batch // _NW
    n_chunks = b_per_w // 128
    n_blocks = b_per_w // _L

    mesh = plsc.VectorSubcoreMesh(core_axis_name="c", subcore_axis_name="s")

    @functools.partial(
        pl.kernel,
        mesh=mesh,
        out_type=jax.ShapeDtypeStruct((batch,), jnp.float32),
        compiler_params=pltpu.CompilerParams(
            needs_layout_passes=False, use_tc_tiling_on_sc=False),
        scratch_types=[
            pltpu.VMEM((n_chunks, 128), jnp.int32),
            pltpu.VMEM((n_chunks, 128), jnp.int32),
            pltpu.VMEM((b_per_w, N_FACTORS), jnp.float32),
            pltpu.VMEM((b_per_w, N_FACTORS), jnp.float32),
            pltpu.VMEM((b_per_w,), jnp.float32),
            pltpu.VMEM((_L * _L,), jnp.float32),
            pltpu.SemaphoreType.DMA,
        ],
    )
    def mf_kernel(x_hbm, user_hbm, movie_hbm, out_hbm,
                  uid_v, mid_v, urows_v, mrows_v, out_v, g_v, sem):
        wid = lax.axis_index("s") * _NC + lax.axis_index("c")
        base = wid * b_per_w

        for j in range(n_chunks):
            pltpu.sync_copy(x_hbm.at[0, pl.ds(base + j * 128, 128)],
                            uid_v.at[j])
            pltpu.sync_copy(x_hbm.at[1, pl.ds(base + j * 128, 128)],
                            mid_v.at[j])

        copies = []
        for j in range(n_chunks):
            copies.append(pltpu.async_copy(
                user_hbm.at[uid_v.at[j]],
                urows_v.at[pl.ds(j * 128, 128)], sem))
            copies.append(pltpu.async_copy(
                movie_hbm.at[mid_v.at[j]],
                mrows_v.at[pl.ds(j * 128, 128)], sem))
        for cp in copies:
            cp.wait()

        lane = lax.iota(jnp.int32, _L)

        def block(blk, carry):
            for r in range(_L):
                row = blk * _L + r
                u0 = urows_v[row, pl.ds(0, _L)]
                u1 = urows_v[row, pl.ds(_L, _L)]
                m0 = mrows_v[row, pl.ds(0, _L)]
                m1 = mrows_v[row, pl.ds(_L, _L)]
                g_v[pl.ds(r * _L, _L)] = u0 * m0 + u1 * m1
            acc = jnp.zeros((_L,), jnp.float32)
            for c in range(_L):
                acc = acc + plsc.load_gather(g_v, [lane * _L + c])
            out_v[pl.ds(blk * _L, _L)] = acc
            return carry

        lax.fori_loop(0, n_blocks, block, 0)
        pltpu.sync_copy(out_v, out_hbm.at[pl.ds(base, b_per_w)])

    return mf_kernel(x, user_emb, movie_emb)


def kernel(x, user_emb, movie_emb):
    return _run(x, user_emb, movie_emb, x.shape[1])


# speedup vs baseline: 4.2386x; 4.2386x over previous
import functools

import jax
import jax.numpy as jnp
from jax import lax
from jax.experimental import pallas as pl
from jax.experimental.pallas import tpu as pltpu
from jax.experimental.pallas import tpu_sc as plsc

N_FACTORS = 32
_LANES = 128
_RING = 8

_info = plsc.get_sparse_core_info()
_NC, _NS, _L = _info.num_cores, _info.num_subcores, _info.num_lanes
_NW = _NC * _NS


@functools.partial(jax.jit, static_argnames=("batch", "n_rows"))
def _run(uidx, midx, user_t, movie_t, utail, mtail, batch, n_rows):
    b_per_w = batch // _NW
    n_groups = b_per_w // _L
    last_blk = ((n_rows - _LANES) // _LANES) * _LANES
    tail_lo = last_blk + _LANES
    tail_start = n_rows - _LANES

    mesh = plsc.VectorSubcoreMesh(core_axis_name="c", subcore_axis_name="s")

    @functools.partial(
        pl.kernel,
        mesh=mesh,
        out_type=jax.ShapeDtypeStruct((batch,), jnp.float32),
        compiler_params=pltpu.CompilerParams(needs_layout_passes=False),
        scratch_types=[
            pltpu.VMEM((b_per_w,), jnp.int32),
            pltpu.VMEM((b_per_w,), jnp.int32),
            pltpu.VMEM((N_FACTORS, _RING * _LANES), jnp.float32),
            pltpu.VMEM((N_FACTORS, _RING * _LANES), jnp.float32),
            pltpu.VMEM((N_FACTORS, _LANES), jnp.float32),
            pltpu.VMEM((N_FACTORS, _LANES), jnp.float32),
            pltpu.VMEM((b_per_w,), jnp.float32),
            pltpu.VMEM((_L * _L,), jnp.float32),
            [pltpu.SemaphoreType.DMA] * _RING,
        ],
    )
    def mf_kernel(uidx_hbm, midx_hbm, user_hbm, movie_hbm,
                  utail_hbm, mtail_hbm, out_hbm,
                  uid_v, mid_v, utile_v, mtile_v, utail_v, mtail_v,
                  out_v, g_v, sems):
        cid = lax.axis_index("c")
        sid = lax.axis_index("s")
        base = pl.multiple_of(sid * (_NC * b_per_w) + cid * b_per_w, _LANES)

        pltpu.sync_copy(uidx_hbm.at[pl.ds(base, b_per_w)], uid_v)
        pltpu.sync_copy(midx_hbm.at[pl.ds(base, b_per_w)], mid_v)
        pltpu.sync_copy(utail_hbm, utail_v)
        pltpu.sync_copy(mtail_hbm, mtail_v)

        lane = lax.iota(jnp.int32, _L)
        blk_mask = jnp.full((_L,), ~(_LANES - 1), jnp.int32)
        blk_max = jnp.full((_L,), last_blk, jnp.int32)

        def blocks_of(p16):
            uids = uid_v[pl.ds(p16, _L)]
            mids = mid_v[pl.ds(p16, _L)]
            ublk = jnp.minimum(uids & blk_mask, blk_max)
            mblk = jnp.minimum(mids & blk_mask, blk_max)
            return uids, mids, ublk, mblk

        def fire_one(ublk_k, mblk_k, slot):
            dst = slot * _LANES
            pltpu.async_copy(
                user_hbm.at[:, pl.ds(pl.multiple_of(ublk_k, _LANES), _LANES)],
                utile_v.at[:, pl.ds(dst, _LANES)], sems[slot])
            pltpu.async_copy(
                movie_hbm.at[:, pl.ds(pl.multiple_of(mblk_k, _LANES), _LANES)],
                mtile_v.at[:, pl.ds(dst, _LANES)], sems[slot])

        def drain(slot):
            off = slot * _LANES
            pltpu.make_async_copy(
                user_hbm.at[:, pl.ds(0, _LANES)],
                utile_v.at[:, pl.ds(off, _LANES)], sems[slot]).wait()
            pltpu.make_async_copy(
                movie_hbm.at[:, pl.ds(0, _LANES)],
                mtile_v.at[:, pl.ds(off, _LANES)], sems[slot]).wait()

        _, _, ub0, mb0 = blocks_of(0)
        for s in range(_RING):
            fire_one(ub0[s], mb0[s], s)

        lo16 = lax.iota(jnp.int32, _L)
        hi16 = lo16 + _L

        def dot_cols(tile_ref, tail_ref, col, tail_col, is_tail):
            sel = jnp.full((_L,), is_tail, jnp.int32) > 0
            a0 = plsc.load_gather(tile_ref, [lo16, col])
            a1 = plsc.load_gather(tile_ref, [hi16, col])
            t0 = plsc.load_gather(tail_ref, [lo16, tail_col])
            t1 = plsc.load_gather(tail_ref, [hi16, tail_col])
            return jnp.where(sel, t0, a0), jnp.where(sel, t1, a1)

        def group(p16, last):
            uids, mids, ublk, mblk = blocks_of(p16)
            ulane = jnp.minimum(uids - ublk, _LANES - 1)
            mlane = jnp.minimum(mids - mblk, _LANES - 1)
            utailc = jnp.maximum(uids - tail_start, 0)
            mtailc = jnp.maximum(mids - tail_start, 0)
            utl = (uids >= tail_lo).astype(jnp.int32)
            mtl = (mids >= tail_lo).astype(jnp.int32)
            nblk = [None, None]
            if last:
                nblk[0], nblk[1] = ublk, mblk
            for j in range(_L):
                slot = j % _RING
                if slot == 0 and not last:
                    _, _, nblk[0], nblk[1] = blocks_of(p16 + j + _RING)
                drain(slot)
                ucol = jnp.full((_L,), slot * _LANES + ulane[j], jnp.int32)
                mcol = jnp.full((_L,), slot * _LANES + mlane[j], jnp.int32)
                utc = jnp.full((_L,), utailc[j], jnp.int32)
                mtc = jnp.full((_L,), mtailc[j], jnp.int32)
                u0, u1 = dot_cols(utile_v, utail_v, ucol, utc, utl[j])
                m0, m1 = dot_cols(mtile_v, mtail_v, mcol, mtc, mtl[j])
                g_v[pl.ds(j * _L, _L)] = u0 * m0 + u1 * m1
                if not last:
                    fire_one(nblk[0][slot], nblk[1][slot], slot)
                elif j < _RING:
                    fire_one(nblk[0][_RING + j], nblk[1][_RING + j], slot)

            acc = jnp.zeros((_L,), jnp.float32)
            for c in range(_L):
                acc = acc + plsc.load_gather(g_v, [lane * _L + c])
            out_v[pl.ds(p16, _L)] = acc

        def body(i, carry):
            group(i * _L, last=False)
            return carry

        lax.fori_loop(0, n_groups - 1, body, 0)
        group(b_per_w - _L, last=True)
        pltpu.sync_copy(out_v, out_hbm.at[pl.ds(base, b_per_w)])

    return mf_kernel(uidx, midx, user_t, movie_t, utail, mtail)


def kernel(x, user_emb, movie_emb):
    n_rows = user_emb.shape[0]
    user_t = user_emb.T
    movie_t = movie_emb.T
    utail = user_t[:, n_rows - _LANES:]
    mtail = movie_t[:, n_rows - _LANES:]
    return _run(x[0], x[1], user_t, movie_t, utail, mtail,
                x.shape[1], n_rows)
